# Initial kernel scaffold; baseline (speedup 1.0000x reference)
import jax
import jax.numpy as jnp
from jax.experimental import pallas as pl


def kernel(x, edge_index, W1, b1, W2, b2, Wmu, bmu, Wls, bls):
    raise NotImplementedError("write your pallas kernel here")


# speedup vs baseline: 5.2756x; 5.2756x over previous
import functools

import jax
import jax.numpy as jnp
from jax import lax
from jax.experimental import pallas as pl
from jax.experimental.pallas import tpu as pltpu
from jax.experimental.pallas import tpu_sc as plsc

N = 10000
E_REAL = 160000
NC = 2
NS = 16
NW = NC * NS
CH = 128
EP = 163840
EPT = EP // NW
NCHUNK = EPT // CH
NPAD = 10240
RPS = NPAD // NS

ADJ_NORM = float(N) * float(N) / ((float(N) * float(N) - float(E_REAL)) * 2.0)

ROW_BLK = 1000
DEC_BM = 1000
DEC_BN = 1024


def _sc_mesh():
    return plsc.VectorSubcoreMesh(
        core_axis_name="c", subcore_axis_name="s", num_cores=NC, num_subcores=NS
    )


def _sc_degree(dst_p, ones16, zrows16):
    @functools.partial(
        pl.kernel,
        out_type=jax.ShapeDtypeStruct((NC, NPAD, 16), jnp.float32),
        mesh=_sc_mesh(),
        scratch_types=[
            pltpu.VMEM((CH,), jnp.int32),
            pltpu.VMEM((CH, 16), jnp.float32),
            pltpu.VMEM_SHARED((NPAD + 1, 16), jnp.float32),
        ],
    )
    def body(dst_hbm, ones_hbm, z_hbm, out_hbm, didx, ones_v, accum):
        c = lax.axis_index("c")
        s = lax.axis_index("s")
        pltpu.sync_copy(z_hbm, accum.at[pl.ds(s * RPS, RPS)])
        pltpu.sync_copy(ones_hbm, ones_v)
        plsc.subcore_barrier()
        ebase = (c * NS + s) * EPT

        def chunk(k, carry):
            base = ebase + k * CH
            pltpu.sync_copy(dst_hbm.at[pl.ds(base, CH)], didx)
            pltpu.sync_copy(ones_v, accum.at[didx], add=True)
            return carry

        lax.fori_loop(0, NCHUNK, chunk, 0)
        plsc.subcore_barrier()
        pltpu.sync_copy(
            accum.at[pl.ds(s * RPS, RPS)], out_hbm.at[c, pl.ds(s * RPS, RPS)]
        )

    return body(dst_p, ones16, zrows16)


def _sc_aggregate(src_p, dst_p, ht, zrows, d):
    @functools.partial(
        pl.kernel,
        out_type=jax.ShapeDtypeStruct((NC, NPAD, d), jnp.float32),
        mesh=_sc_mesh(),
        scratch_types=[
            pltpu.VMEM((CH,), jnp.int32),
            pltpu.VMEM((CH,), jnp.int32),
            pltpu.VMEM((CH, d), jnp.float32),
            pltpu.VMEM_SHARED((NPAD + 1, d), jnp.float32),
            pltpu.SemaphoreType.DMA,
        ],
    )
    def body(src_hbm, dst_hbm, ht_hbm, z_hbm, out_hbm, sidx, didx, rows, accum, sem):
        c = lax.axis_index("c")
        s = lax.axis_index("s")
        pltpu.sync_copy(z_hbm, accum.at[pl.ds(s * RPS, RPS)])
        plsc.subcore_barrier()
        ebase = (c * NS + s) * EPT

        def chunk(k, carry):
            base = ebase + k * CH
            pltpu.sync_copy(src_hbm.at[pl.ds(base, CH)], sidx)
            pltpu.sync_copy(dst_hbm.at[pl.ds(base, CH)], didx)
            pltpu.async_copy(ht_hbm.at[sidx], rows, sem).wait()
            pltpu.sync_copy(rows, accum.at[didx], add=True)
            return carry

        lax.fori_loop(0, NCHUNK, chunk, 0)
        plsc.subcore_barrier()
        pltpu.sync_copy(
            accum.at[pl.ds(s * RPS, RPS)], out_hbm.at[c, pl.ds(s * RPS, RPS)]
        )

    return body(src_p, dst_p, ht, zrows)


def _dis(degp_ref):
    d = 1.0 + degp_ref[0, :, 0:1] + degp_ref[1, :, 0:1]
    return lax.rsqrt(d)


def _tc_prep(x, w1, degp):
    def body(x_ref, w_ref, degp_ref, out_ref):
        dis = _dis(degp_ref)
        out_ref[...] = dis * jnp.dot(
            x_ref[...], w_ref[...], preferred_element_type=jnp.float32
        )

    grid = (N // ROW_BLK,)
    return pl.pallas_call(
        body,
        grid=grid,
        in_specs=[
            pl.BlockSpec((ROW_BLK, 128), lambda i: (i, 0)),
            pl.BlockSpec((128, 128), lambda i: (0, 0)),
            pl.BlockSpec((NC, ROW_BLK, 16), lambda i: (0, i, 0)),
        ],
        out_specs=pl.BlockSpec((ROW_BLK, 128), lambda i: (i, 0)),
        out_shape=jax.ShapeDtypeStruct((N, 128), jnp.float32),
    )(x, w1, degp)


def _tc_layer(p, ht_prev, degp, b, w, d_out):
    def body(p_ref, ht_ref, degp_ref, b_ref, w_ref, out_ref):
        dis = _dis(degp_ref)
        pre = dis * (p_ref[0] + p_ref[1] + ht_ref[...]) + b_ref[...]
        h = jnp.maximum(pre, 0.0)
        out_ref[...] = dis * jnp.dot(
            h, w_ref[...], preferred_element_type=jnp.float32
        )

    grid = (N // ROW_BLK,)
    return pl.pallas_call(
        body,
        grid=grid,
        in_specs=[
            pl.BlockSpec((NC, ROW_BLK, 128), lambda i: (0, i, 0)),
            pl.BlockSpec((ROW_BLK, 128), lambda i: (i, 0)),
            pl.BlockSpec((NC, ROW_BLK, 16), lambda i: (0, i, 0)),
            pl.BlockSpec((1, 128), lambda i: (0, 0)),
            pl.BlockSpec((128, d_out), lambda i: (0, 0)),
        ],
        out_specs=pl.BlockSpec((ROW_BLK, d_out), lambda i: (i, 0)),
        out_shape=jax.ShapeDtypeStruct((N, d_out), jnp.float32),
    )(p, ht_prev, degp, b, w)


def _tc_final(p, zt, degp, bmu):
    def body(p_ref, zt_ref, degp_ref, b_ref, out_ref):
        dis = _dis(degp_ref)
        full = dis * (p_ref[0] + p_ref[1] + zt_ref[...])
        out_ref[...] = full[:, 0:32] + b_ref[...]

    grid = (N // ROW_BLK,)
    return pl.pallas_call(
        body,
        grid=grid,
        in_specs=[
            pl.BlockSpec((NC, ROW_BLK, 128), lambda i: (0, i, 0)),
            pl.BlockSpec((ROW_BLK, 128), lambda i: (i, 0)),
            pl.BlockSpec((NC, ROW_BLK, 16), lambda i: (0, i, 0)),
            pl.BlockSpec((1, 32), lambda i: (0, 0)),
        ],
        out_specs=pl.BlockSpec((ROW_BLK, 32), lambda i: (i, 0)),
        out_shape=jax.ShapeDtypeStruct((N, 32), jnp.float32),
    )(p, zt, degp, bmu)


def _tc_decode(z):
    def body(zi_ref, zj_ref, out_ref):
        acc = lax.dot_general(
            zi_ref[...],
            zj_ref[...],
            (((1,), (1,)), ((), ())),
            preferred_element_type=jnp.float32,
        )
        out_ref[...] = jax.nn.sigmoid(acc) * ADJ_NORM

    grid = (N // DEC_BM, pl.cdiv(N, DEC_BN))
    return pl.pallas_call(
        body,
        grid=grid,
        in_specs=[
            pl.BlockSpec((DEC_BM, 32), lambda i, j: (i, 0)),
            pl.BlockSpec((DEC_BN, 32), lambda i, j: (j, 0)),
        ],
        out_specs=pl.BlockSpec((DEC_BM, DEC_BN), lambda i, j: (i, j)),
        out_shape=jax.ShapeDtypeStruct((N, N), jnp.float32),
    )(z, z)


def kernel(x, edge_index, W1, b1, W2, b2, Wmu, bmu, Wls, bls):
    src = edge_index[0]
    dst = edge_index[1]
    pad = EP - E_REAL
    src_p = jnp.concatenate([src, jnp.zeros((pad,), jnp.int32)])
    dst_p = jnp.concatenate([dst, jnp.full((pad,), NPAD, jnp.int32)])

    ones16 = jnp.ones((CH, 16), jnp.float32)
    z16 = jnp.zeros((RPS, 16), jnp.float32)
    z128 = jnp.zeros((RPS, 128), jnp.float32)
    z32 = jnp.zeros((RPS, 32), jnp.float32)

    wmu128 = jnp.zeros((128, 128), jnp.float32).at[:, 0:32].set(Wmu)

    degp = _sc_degree(dst_p, ones16, z16)
    ht1 = _tc_prep(x, W1, degp)
    p1 = _sc_aggregate(src_p, dst_p, ht1, z128, 128)
    ht2 = _tc_layer(p1, ht1, degp, b1.reshape(1, 128), W2, 128)
    p2 = _sc_aggregate(src_p, dst_p, ht2, z128, 128)
    zt = _tc_layer(p2, ht2, degp, b2.reshape(1, 128), wmu128, 128)
    p3 = _sc_aggregate(src_p, dst_p, zt, z128, 128)
    z = _tc_final(p3, zt, degp, bmu.reshape(1, 32))
    return _tc_decode(z)
